# baseline (speedup 1.0000x reference)
import functools

import jax
import jax.numpy as jnp
from jax import lax
from jax.experimental import pallas as pl
from jax.experimental.pallas import tpu as pltpu
from jax.experimental.pallas import tpu_sc as plsc

TBL = 1000
TBL_PAD = 1024
IN_DIM = 256
D = 1024
B = 16384

NC = 2
NS = 16
NW = NC * NS
B_PER_W = B // NW
CHUNK = 8
NCH = B_PER_W // CHUNK
NBUF = 8


def _sigmoid(x):
    return 1.0 / (1.0 + jnp.exp(-x))


def _mlp_table_body(e_ref, w1_ref, b1_ref, w2_ref, b2_ref, y_ref):
    e = jnp.concatenate(
        [e_ref[...], jnp.zeros((TBL_PAD - TBL, IN_DIM), jnp.float32)], axis=0)
    h = jnp.dot(e, w1_ref[...], preferred_element_type=jnp.float32)
    h = h + b1_ref[...]
    h = h * _sigmoid(h)
    y = jnp.dot(h, w2_ref[...], preferred_element_type=jnp.float32)
    y = y + b2_ref[...]
    y_ref[...] = y * _sigmoid(y)


def _mlp_table(e, W1, b1, W2, b2):
    return pl.pallas_call(
        _mlp_table_body,
        out_shape=jax.ShapeDtypeStruct((TBL_PAD, D), jnp.float32),
    )(e, W1, b1.reshape(1, D), W2, b2.reshape(1, D))


_sc_mesh = plsc.VectorSubcoreMesh(core_axis_name="c", subcore_axis_name="s")


@functools.partial(
    pl.kernel,
    out_type=jax.ShapeDtypeStruct((B, D), jnp.float32),
    mesh=_sc_mesh,
    scratch_types=(
        [pltpu.VMEM((NCH, CHUNK), jnp.int32)]
        + [pltpu.VMEM((CHUNK, D), jnp.float32)] * NBUF
        + [pltpu.SemaphoreType.DMA] * (2 * NBUF)
    ),
)
def _sc_gather(table_hbm, idx_hbm, out_hbm, idx_v, *bufs_and_sems):
    bufs = bufs_and_sems[:NBUF]
    gsem = bufs_and_sems[NBUF:2 * NBUF]
    psem = bufs_and_sems[2 * NBUF:]
    wid = lax.axis_index("s") * NC + lax.axis_index("c")
    base = wid * B_PER_W
    pltpu.sync_copy(idx_hbm.at[wid], idx_v)
    gets = [None] * NBUF
    puts = [None] * NBUF

    def wait_put(slot):
        if puts[slot] is not None:
            puts[slot].wait()
            puts[slot] = None

    for c in range(min(4, NCH)):
        gets[c] = pltpu.async_copy(
            table_hbm.at[idx_v.at[c]], bufs[c], gsem[c])
    for c in range(NCH):
        b = c % NBUF
        gets[b].wait()
        c2 = c + 4
        if c2 < NCH:
            b2 = c2 % NBUF
            wait_put(b2)
            gets[b2] = pltpu.async_copy(
                table_hbm.at[idx_v.at[c2]], bufs[b2], gsem[b2])
        puts[b] = pltpu.async_copy(
            bufs[b], out_hbm.at[pl.ds(base + c * CHUNK, CHUNK)], psem[b])
    for b in range(NBUF):
        wait_put(b)


def kernel(diffusion_step, embedding, W1, b1, W2, b2):
    y = _mlp_table(embedding, W1, b1, W2, b2)
    idx = diffusion_step.reshape(NW, NCH, CHUNK)
    return _sc_gather(y, idx)


# speedup vs baseline: 1.0221x; 1.0221x over previous
import functools

import jax
import jax.numpy as jnp
from jax import lax
from jax.experimental import pallas as pl
from jax.experimental.pallas import tpu as pltpu
from jax.experimental.pallas import tpu_sc as plsc

TBL = 1000
TBL_PAD = 1024
IN_DIM = 256
D = 1024
B = 16384

NC = 2
NS = 16
NW = NC * NS
B_PER_W = B // NW
CHUNK = 16
NCH = B_PER_W // CHUNK
NBUF = 7


def _sigmoid(x):
    return 1.0 / (1.0 + jnp.exp(-x))


def _mlp_table_body(e_ref, w1_ref, b1_ref, w2_ref, b2_ref, y_ref):
    e = jnp.concatenate(
        [e_ref[...], jnp.zeros((TBL_PAD - TBL, IN_DIM), jnp.float32)], axis=0)
    h = jnp.dot(e, w1_ref[...], preferred_element_type=jnp.float32)
    h = h + b1_ref[...]
    h = h * _sigmoid(h)
    y = jnp.dot(h, w2_ref[...], preferred_element_type=jnp.float32)
    y = y + b2_ref[...]
    y_ref[...] = y * _sigmoid(y)


def _mlp_table(e, W1, b1, W2, b2):
    return pl.pallas_call(
        _mlp_table_body,
        out_shape=jax.ShapeDtypeStruct((TBL_PAD, D), jnp.float32),
    )(e, W1, b1.reshape(1, D), W2, b2.reshape(1, D))


_sc_mesh = plsc.VectorSubcoreMesh(core_axis_name="c", subcore_axis_name="s")


@functools.partial(
    pl.kernel,
    out_type=jax.ShapeDtypeStruct((B, D), jnp.float32),
    mesh=_sc_mesh,
    scratch_types=(
        [pltpu.VMEM((NCH, CHUNK), jnp.int32)]
        + [pltpu.VMEM((CHUNK, D), jnp.float32)] * NBUF
        + [pltpu.SemaphoreType.DMA] * (2 * NBUF)
    ),
)
def _sc_gather(table_hbm, idx_hbm, out_hbm, idx_v, *bufs_and_sems):
    bufs = bufs_and_sems[:NBUF]
    gsem = bufs_and_sems[NBUF:2 * NBUF]
    psem = bufs_and_sems[2 * NBUF:]
    wid = lax.axis_index("s") * NC + lax.axis_index("c")
    base = wid * B_PER_W
    pltpu.sync_copy(idx_hbm.at[wid], idx_v)
    gets = [None] * NBUF
    puts = [None] * NBUF

    def wait_put(slot):
        if puts[slot] is not None:
            puts[slot].wait()
            puts[slot] = None

    for c in range(min(4, NCH)):
        gets[c] = pltpu.async_copy(
            table_hbm.at[idx_v.at[c]], bufs[c], gsem[c])
    for c in range(NCH):
        b = c % NBUF
        gets[b].wait()
        c2 = c + 4
        if c2 < NCH:
            b2 = c2 % NBUF
            wait_put(b2)
            gets[b2] = pltpu.async_copy(
                table_hbm.at[idx_v.at[c2]], bufs[b2], gsem[b2])
        puts[b] = pltpu.async_copy(
            bufs[b], out_hbm.at[pl.ds(base + c * CHUNK, CHUNK)], psem[b])
    for b in range(NBUF):
        wait_put(b)


def kernel(diffusion_step, embedding, W1, b1, W2, b2):
    y = _mlp_table(embedding, W1, b1, W2, b2)
    idx = diffusion_step.reshape(NW, NCH, CHUNK)
    return _sc_gather(y, idx)


# speedup vs baseline: 1.0227x; 1.0006x over previous
import functools

import jax
import jax.numpy as jnp
from jax import lax
from jax.experimental import pallas as pl
from jax.experimental.pallas import tpu as pltpu
from jax.experimental.pallas import tpu_sc as plsc

TBL = 1000
TBL_PAD = 1024
IN_DIM = 256
D = 1024
B = 16384

NC = 2
NS = 16
NW = NC * NS
B_PER_W = B // NW
CHUNK = 16
NCH = B_PER_W // CHUNK
NBUF = 7


def _sigmoid(x):
    return 1.0 / (1.0 + jnp.exp(-x))


def _mlp_table_body(e_ref, w1_ref, b1_ref, w2_ref, b2_ref, y_ref):
    e = jnp.concatenate(
        [e_ref[...], jnp.zeros((TBL_PAD - TBL, IN_DIM), jnp.float32)], axis=0)
    h = jnp.dot(e.astype(jnp.bfloat16), w1_ref[...].astype(jnp.bfloat16),
                preferred_element_type=jnp.float32)
    h = h + b1_ref[...]
    h = h * _sigmoid(h)
    y = jnp.dot(h.astype(jnp.bfloat16), w2_ref[...].astype(jnp.bfloat16),
                preferred_element_type=jnp.float32)
    y = y + b2_ref[...]
    y_ref[...] = y * _sigmoid(y)


def _mlp_table(e, W1, b1, W2, b2):
    return pl.pallas_call(
        _mlp_table_body,
        out_shape=jax.ShapeDtypeStruct((TBL_PAD, D), jnp.float32),
    )(e, W1, b1.reshape(1, D), W2, b2.reshape(1, D))


_sc_mesh = plsc.VectorSubcoreMesh(core_axis_name="c", subcore_axis_name="s")


@functools.partial(
    pl.kernel,
    out_type=jax.ShapeDtypeStruct((B, D), jnp.float32),
    mesh=_sc_mesh,
    scratch_types=(
        [pltpu.VMEM((NCH, CHUNK), jnp.int32)]
        + [pltpu.VMEM((CHUNK, D), jnp.float32)] * NBUF
        + [pltpu.SemaphoreType.DMA] * (2 * NBUF)
    ),
)
def _sc_gather(table_hbm, idx_hbm, out_hbm, idx_v, *bufs_and_sems):
    bufs = bufs_and_sems[:NBUF]
    gsem = bufs_and_sems[NBUF:2 * NBUF]
    psem = bufs_and_sems[2 * NBUF:]
    wid = lax.axis_index("s") * NC + lax.axis_index("c")
    base = wid * B_PER_W
    pltpu.sync_copy(idx_hbm.at[wid], idx_v)
    gets = [None] * NBUF
    puts = [None] * NBUF

    def wait_put(slot):
        if puts[slot] is not None:
            puts[slot].wait()
            puts[slot] = None

    for c in range(min(4, NCH)):
        gets[c] = pltpu.async_copy(
            table_hbm.at[idx_v.at[c]], bufs[c], gsem[c])
    for c in range(NCH):
        b = c % NBUF
        gets[b].wait()
        c2 = c + 4
        if c2 < NCH:
            b2 = c2 % NBUF
            wait_put(b2)
            gets[b2] = pltpu.async_copy(
                table_hbm.at[idx_v.at[c2]], bufs[b2], gsem[b2])
        puts[b] = pltpu.async_copy(
            bufs[b], out_hbm.at[pl.ds(base + c * CHUNK, CHUNK)], psem[b])
    for b in range(NBUF):
        wait_put(b)


def kernel(diffusion_step, embedding, W1, b1, W2, b2):
    y = _mlp_table(embedding, W1, b1, W2, b2)
    idx = diffusion_step.reshape(NW, NCH, CHUNK)
    return _sc_gather(y, idx)


# speedup vs baseline: 1.0241x; 1.0014x over previous
import functools

import jax
import jax.numpy as jnp
from jax import lax
from jax.experimental import pallas as pl
from jax.experimental.pallas import tpu as pltpu
from jax.experimental.pallas import tpu_sc as plsc

TBL = 1000
TBL_PAD = 1024
IN_DIM = 256
D = 1024
B = 16384

NC = 2
NS = 16
NW = NC * NS
B_PER_W = B // NW
CHUNK = 16
NCH = B_PER_W // CHUNK
NBUF = 7


def _sigmoid(x):
    return 1.0 / (1.0 + jnp.exp(-x))


def _mlp_table_body(e_ref, w1_ref, b1_ref, w2_ref, b2_ref, y_ref):
    e = jnp.concatenate(
        [e_ref[...], jnp.zeros((TBL_PAD - TBL, IN_DIM), jnp.float32)], axis=0)
    h = jnp.dot(e, w1_ref[...], preferred_element_type=jnp.float32)
    h = h + b1_ref[...]
    h = h * _sigmoid(h)
    y = jnp.dot(h, w2_ref[...], preferred_element_type=jnp.float32)
    y = y + b2_ref[...]
    y_ref[...] = y * _sigmoid(y)


def _mlp_table(e, W1, b1, W2, b2):
    return pl.pallas_call(
        _mlp_table_body,
        out_shape=jax.ShapeDtypeStruct((TBL_PAD, D), jnp.float32),
    )(e, W1, b1.reshape(1, D), W2, b2.reshape(1, D))


_sc_mesh = plsc.VectorSubcoreMesh(core_axis_name="c", subcore_axis_name="s")


@functools.partial(
    pl.kernel,
    out_type=jax.ShapeDtypeStruct((B, D), jnp.float32),
    mesh=_sc_mesh,
    scratch_types=(
        [pltpu.VMEM((NCH, CHUNK), jnp.int32)]
        + [pltpu.VMEM((CHUNK, D), jnp.float32)] * NBUF
        + [pltpu.SemaphoreType.DMA] * (2 * NBUF)
    ),
)
def _sc_gather(table_hbm, idx_hbm, out_hbm, idx_v, *bufs_and_sems):
    bufs = bufs_and_sems[:NBUF]
    gsem = bufs_and_sems[NBUF:2 * NBUF]
    psem = bufs_and_sems[2 * NBUF:]
    wid = lax.axis_index("s") * NC + lax.axis_index("c")
    base = wid * B_PER_W
    pltpu.sync_copy(idx_hbm.at[wid], idx_v)
    gets = [None] * NBUF
    puts = [None] * NBUF

    def wait_put(slot):
        if puts[slot] is not None:
            puts[slot].wait()
            puts[slot] = None

    for c in range(min(4, NCH)):
        gets[c] = pltpu.async_copy(
            table_hbm.at[idx_v.at[c]], bufs[c], gsem[c])
    for c in range(NCH):
        b = c % NBUF
        gets[b].wait()
        c2 = c + 4
        if c2 < NCH:
            b2 = c2 % NBUF
            wait_put(b2)
            gets[b2] = pltpu.async_copy(
                table_hbm.at[idx_v.at[c2]], bufs[b2], gsem[b2])
        puts[b] = pltpu.async_copy(
            bufs[b], out_hbm.at[pl.ds(base + c * CHUNK, CHUNK)], psem[b])
    for b in range(NBUF):
        wait_put(b)


def kernel(diffusion_step, embedding, W1, b1, W2, b2):
    y = _mlp_table(embedding, W1, b1, W2, b2)
    idx = diffusion_step.reshape(NW, NCH, CHUNK)
    return _sc_gather(y, idx)
